# baseline (speedup 1.0000x reference)
import functools

import jax
import jax.numpy as jnp
from jax import lax
from jax.experimental import pallas as pl
from jax.experimental.pallas import tpu as pltpu
from jax.experimental.pallas import tpu_sc as plsc

_VOCAB = 1000000
_EMB = 64
_B = 16384
_L = 20
_NC = 2
_NS = 16
_NW = _NC * _NS
_BPW = _B // _NW
_CB = 16
_NSUB = _BPW // _CB
_ROWS = _CB * _L
_GCH = 64
_NG = _ROWS // _GCH
_DC = 8

_mesh = plsc.VectorSubcoreMesh(
    core_axis_name="c", subcore_axis_name="s", num_cores=_NC, num_subcores=_NS
)


@functools.partial(
    pl.kernel,
    out_type=jax.ShapeDtypeStruct((_B * _L,), jnp.float32),
    mesh=_mesh,
    compiler_params=pltpu.CompilerParams(
        use_tc_tiling_on_sc=True, needs_layout_passes=False),
    scratch_types=[
        pltpu.VMEM((_BPW,), jnp.int32),
        pltpu.VMEM((_BPW,), jnp.int32),
        pltpu.VMEM((_BPW * _L,), jnp.int32),
        pltpu.VMEM((_BPW * _L,), jnp.int32),
        pltpu.VMEM((2 * _CB, 2 * _EMB), jnp.float32),
        pltpu.VMEM((2 * _ROWS, 2 * _EMB), jnp.float32),
        pltpu.VMEM((_BPW * _L,), jnp.float32),
        pltpu.SemaphoreType.DMA,
    ],
)
def _sc_skipgram(central_hbm, ctxidx_hbm, win_hbm, wout_hbm, out_hbm,
                 cidx_v, cpar_v, xidx_v, xpar_v, hid_v, ctx_v, out_v, sem):
    wid = lax.axis_index("s") * _NC + lax.axis_index("c")
    base = wid * _BPW
    pltpu.sync_copy(central_hbm.at[pl.ds(base, _BPW)], cidx_v)
    pltpu.sync_copy(ctxidx_hbm.at[pl.ds(base * _L, _BPW * _L)], xidx_v)

    def split_idx(i, _):
        v = xidx_v[pl.ds(i * 16, 16)]
        xpar_v[pl.ds(i * 16, 16)] = (v & 1) * _EMB
        xidx_v[pl.ds(i * 16, 16)] = v >> 1
        return 0

    lax.fori_loop(0, _BPW * _L // 16, split_idx, 0)

    def split_cidx(i, _):
        v = cidx_v[pl.ds(i * 16, 16)]
        cpar_v[pl.ds(i * 16, 16)] = (v & 1) * _EMB
        cidx_v[pl.ds(i * 16, 16)] = v >> 1
        return 0

    lax.fori_loop(0, _BPW // 16, split_cidx, 0)

    lanes = lax.iota(jnp.int32, 16)

    def fire(sc, hofs, cofs):
        pltpu.async_copy(
            win_hbm.at[cidx_v.at[pl.ds(sc * _CB, _CB)]],
            hid_v.at[pl.ds(hofs, _CB)], sem)
        for j in range(_NG):
            pltpu.async_copy(
                wout_hbm.at[xidx_v.at[pl.ds(sc * _ROWS + j * _GCH, _GCH)]],
                ctx_v.at[pl.ds(cofs + j * _GCH, _GCH)], sem)

    fire(0, 0, 0)

    def sub(sc, _):
        p = sc % 2
        hofs = p * _CB
        cofs = p * _ROWS
        pltpu.make_async_copy(
            win_hbm.at[pl.ds(0, _CB)], hid_v.at[pl.ds(hofs, _CB)], sem).wait()
        pltpu.make_async_copy(
            wout_hbm.at[pl.ds(0, _ROWS)], ctx_v.at[pl.ds(cofs, _ROWS)],
            sem).wait()

        @pl.when(sc < _NSUB - 1)
        def _():
            q = (sc + 1) % 2
            fire(sc + 1, q * _CB, q * _ROWS)

        hrow = lanes + hofs
        crow = lanes * _L + cofs
        pbase = sc * _ROWS + lanes * _L
        hpar = plsc.load_gather(cpar_v, [sc * _CB + lanes])
        xpars = [plsc.load_gather(xpar_v, [pbase + l]) for l in range(_L)]

        def dstep(dc, accs):
            d0 = dc * _DC
            hcols = [hpar + (d0 + dd) for dd in range(_DC)]
            hvs = [plsc.load_gather(hid_v, [hrow, hcols[dd]])
                   for dd in range(_DC)]
            out = []
            for l in range(_L):
                rowv = crow + l
                acc = accs[l]
                for dd in range(_DC):
                    cv = plsc.load_gather(ctx_v,
                                          [rowv, xpars[l] + (d0 + dd)])
                    acc = acc + cv * hvs[dd]
                out.append(acc)
            return tuple(out)

        accs = lax.fori_loop(
            0, _EMB // _DC, dstep,
            tuple(jnp.zeros(16, jnp.float32) for _ in range(_L)))

        for l in range(_L):
            sig = 1.0 / (1.0 + jnp.exp(-accs[l]))
            plsc.store_scatter(out_v, [pbase + l], sig)
        return 0

    lax.fori_loop(0, _NSUB, sub, 0)
    pltpu.sync_copy(out_v, out_hbm.at[pl.ds(base * _L, _BPW * _L)])


def kernel(central_items, context_items, W_in, W_out):
    out = _sc_skipgram(
        central_items.astype(jnp.int32),
        context_items.reshape(-1).astype(jnp.int32),
        W_in.reshape(_VOCAB // 2, 2 * _EMB),
        W_out.reshape(_VOCAB // 2, 2 * _EMB),
    )
    return out.reshape(_B, _L)


# speedup vs baseline: 1.0610x; 1.0610x over previous
import functools

import jax
import jax.numpy as jnp
from jax import lax
from jax.experimental import pallas as pl
from jax.experimental.pallas import tpu as pltpu
from jax.experimental.pallas import tpu_sc as plsc

_EMB = 64
_B = 16384
_L = 20
_NC = 2
_NS = 16
_NW = _NC * _NS
_BPW = _B // _NW
_CB = 16
_NSUB = _BPW // _CB
_ROWS = _CB * _L
_DC = 8

_mesh = plsc.VectorSubcoreMesh(
    core_axis_name="c", subcore_axis_name="s", num_cores=_NC, num_subcores=_NS
)
_params = pltpu.CompilerParams(
    use_tc_tiling_on_sc=True, needs_layout_passes=False,
    disable_bounds_checks=True)


@functools.partial(
    pl.kernel,
    out_type=jax.ShapeDtypeStruct((_B * _EMB,), jnp.float32),
    mesh=_mesh,
    compiler_params=_params,
    scratch_types=[
        pltpu.VMEM((_BPW,), jnp.int32),
        pltpu.VMEM((2 * _EMB, 128), jnp.float32),
        pltpu.VMEM((_CB * _EMB,), jnp.float32),
        pltpu.SemaphoreType.DMA,
    ],
)
def _sc_hidden(central_hbm, wint_hbm, hid_hbm, cidx_v, blk_v, rows_v, sem):
    wid = lax.axis_index("s") * _NC + lax.axis_index("c")
    base = wid * _BPW
    pltpu.sync_copy(central_hbm.at[pl.ds(base, _BPW)], cidx_v)

    lanes = lax.iota(jnp.int32, 16)
    dvecs = [lanes + 16 * j for j in range(_EMB // 16)]

    def fire(v, bofs):
        blk = pl.multiple_of((v >> 7) << 7, 128)
        pltpu.async_copy(wint_hbm.at[:, pl.ds(blk, 128)],
                         blk_v.at[pl.ds(bofs, _EMB)], sem)

    first = cidx_v[pl.ds(0, 16)]
    fire(first[0], 0)

    def group(g, _):
        vv = cidx_v[pl.ds(g * _CB, _CB)]
        nxt = cidx_v[pl.ds((g + 1) % _NSUB * _CB, _CB)]
        for j in range(_CB):
            p = (g * _CB + j) % 2
            bofs = p * _EMB
            pltpu.make_async_copy(
                wint_hbm.at[:, pl.ds(0, 128)], blk_v.at[pl.ds(bofs, _EMB)],
                sem).wait()
            if j < _CB - 1:
                fire(vv[j + 1], (1 - p) * _EMB)
            else:
                @pl.when(g < _NSUB - 1)
                def _():
                    fire(nxt[0], (1 - p) * _EMB)
            m = jnp.full((16,), vv[j] & 127, jnp.int32)
            for j4 in range(_EMB // 16):
                col = plsc.load_gather(blk_v, [bofs + dvecs[j4], m])
                rows_v[pl.ds(j * _EMB + j4 * 16, 16)] = col
        pltpu.sync_copy(
            rows_v, hid_hbm.at[pl.ds((base + g * _CB) * _EMB, _CB * _EMB)])
        return 0

    lax.fori_loop(0, _NSUB, group, 0)


@functools.partial(
    pl.kernel,
    out_type=jax.ShapeDtypeStruct((_B * _L,), jnp.float32),
    mesh=_mesh,
    compiler_params=_params,
    scratch_types=[
        pltpu.VMEM((_BPW * _L,), jnp.int32),
        pltpu.VMEM((2 * _CB * _EMB,), jnp.float32),
        pltpu.VMEM((2 * _ROWS, _EMB), jnp.float32),
        pltpu.VMEM((_BPW * _L,), jnp.float32),
        pltpu.SemaphoreType.DMA,
    ],
)
def _sc_logits(ctxidx_hbm, wout_hbm, hid_hbm, out_hbm,
               xidx_v, hid_v, ctx_v, out_v, sem):
    wid = lax.axis_index("s") * _NC + lax.axis_index("c")
    base = wid * _BPW
    pltpu.sync_copy(ctxidx_hbm.at[pl.ds(base * _L, _BPW * _L)], xidx_v)

    lanes = lax.iota(jnp.int32, 16)

    def fire(sc, hofs, cofs):
        pltpu.async_copy(
            hid_hbm.at[pl.ds((base + sc * _CB) * _EMB, _CB * _EMB)],
            hid_v.at[pl.ds(hofs * _EMB, _CB * _EMB)], sem)
        for p in range(_ROWS // 16):
            xv = xidx_v[pl.ds(sc * _ROWS + p * 16, 16)]
            for j in range(16):
                pltpu.async_copy(
                    wout_hbm.at[pl.ds(xv[j], 1)],
                    ctx_v.at[pl.ds(cofs + p * 16 + j, 1)], sem)

    fire(0, 0, 0)

    def sub(sc, _):
        p = sc % 2
        hofs = p * _CB
        cofs = p * _ROWS
        pltpu.make_async_copy(
            hid_hbm.at[pl.ds(0, _CB * _EMB)],
            hid_v.at[pl.ds(hofs * _EMB, _CB * _EMB)], sem).wait()
        pltpu.make_async_copy(
            wout_hbm.at[pl.ds(0, _ROWS)], ctx_v.at[pl.ds(cofs, _ROWS)],
            sem).wait()

        @pl.when(sc < _NSUB - 1)
        def _():
            q = (sc + 1) % 2
            fire(sc + 1, q * _CB, q * _ROWS)

        hbase = (lanes + hofs) * _EMB
        crow = lanes * _L + cofs
        obase = sc * _ROWS + lanes * _L

        for l in range(_L):
            rowv = crow + l

            def dstep(dc, acc):
                d0 = dc * _DC
                for dd in range(_DC):
                    dcol = jnp.full((16,), d0 + dd, jnp.int32)
                    cv = plsc.load_gather(ctx_v, [rowv, dcol])
                    hv = plsc.load_gather(hid_v, [hbase + (d0 + dd)])
                    acc = acc + cv * hv
                return acc

            acc = lax.fori_loop(0, _EMB // _DC, dstep,
                                jnp.zeros(16, jnp.float32))
            sig = 1.0 / (1.0 + jnp.exp(-acc))
            plsc.store_scatter(out_v, [obase + l], sig)
        return 0

    lax.fori_loop(0, _NSUB, sub, 0)
    pltpu.sync_copy(out_v, out_hbm.at[pl.ds(base * _L, _BPW * _L)])


def kernel(central_items, context_items, W_in, W_out):
    hidden = _sc_hidden(central_items.astype(jnp.int32), W_in.T)
    out = _sc_logits(
        context_items.reshape(-1).astype(jnp.int32),
        W_out,
        hidden,
    )
    return out.reshape(_B, _L)


# speedup vs baseline: 1.5280x; 1.4401x over previous
import functools

import jax
import jax.numpy as jnp
from jax import lax
from jax.experimental import pallas as pl
from jax.experimental.pallas import tpu as pltpu
from jax.experimental.pallas import tpu_sc as plsc

_EMB = 64
_B = 16384
_L = 20
_NC = 2
_NS = 16
_NW = _NC * _NS
_BPW = _B // _NW
_CB = 16
_NSUB = _BPW // _CB
_ROWS = _CB * _L
_DC = 8

_mesh = plsc.VectorSubcoreMesh(
    core_axis_name="c", subcore_axis_name="s", num_cores=_NC, num_subcores=_NS
)
_params = pltpu.CompilerParams(
    use_tc_tiling_on_sc=True, needs_layout_passes=False,
    disable_bounds_checks=True)


@functools.partial(
    pl.kernel,
    out_type=jax.ShapeDtypeStruct((_B * _EMB,), jnp.float32),
    mesh=_mesh,
    compiler_params=_params,
    scratch_types=[
        pltpu.VMEM((_BPW,), jnp.int32),
        pltpu.VMEM((4 * _EMB, 128), jnp.float32),
        pltpu.VMEM((_CB * _EMB,), jnp.float32),
        pltpu.SemaphoreType.DMA,
    ],
)
def _sc_hidden(central_hbm, wint_hbm, hid_hbm, cidx_v, blk_v, rows_v, sem):
    wid = lax.axis_index("s") * _NC + lax.axis_index("c")
    base = wid * _BPW
    pltpu.sync_copy(central_hbm.at[pl.ds(base, _BPW)], cidx_v)

    lanes = lax.iota(jnp.int32, 16)
    dvecs = [lanes + 16 * j for j in range(_EMB // 16)]

    def fire(v, bofs):
        blk = pl.multiple_of((v >> 7) << 7, 128)
        pltpu.async_copy(wint_hbm.at[:, pl.ds(blk, 128)],
                         blk_v.at[pl.ds(bofs, _EMB)], sem)

    first = cidx_v[pl.ds(0, 16)]
    for k in range(4):
        fire(first[k], k * _EMB)

    def group(g, _):
        vv = cidx_v[pl.ds(g * _CB, _CB)]
        nxt = cidx_v[pl.ds((g + 1) % _NSUB * _CB, _CB)]
        for j in range(_CB):
            slot = (g * _CB + j) % 4
            bofs = slot * _EMB
            pltpu.make_async_copy(
                wint_hbm.at[:, pl.ds(0, 128)], blk_v.at[pl.ds(bofs, _EMB)],
                sem).wait()
            if j + 4 < _CB:
                fire(vv[j + 4], bofs)
            else:
                @pl.when(g < _NSUB - 1)
                def _():
                    fire(nxt[j + 4 - _CB], bofs)
            m = jnp.full((16,), vv[j] & 127, jnp.int32)
            for j4 in range(_EMB // 16):
                col = plsc.load_gather(blk_v, [bofs + dvecs[j4], m])
                rows_v[pl.ds(j * _EMB + j4 * 16, 16)] = col
        pltpu.sync_copy(
            rows_v, hid_hbm.at[pl.ds((base + g * _CB) * _EMB, _CB * _EMB)])
        return 0

    lax.fori_loop(0, _NSUB, group, 0)


@functools.partial(
    pl.kernel,
    out_type=jax.ShapeDtypeStruct((_B * _L,), jnp.float32),
    mesh=_mesh,
    compiler_params=_params,
    scratch_types=[
        pltpu.VMEM((_BPW * _L,), jnp.int32),
        pltpu.VMEM((2 * _CB * _EMB,), jnp.float32),
        pltpu.VMEM((2 * _ROWS, _EMB), jnp.float32),
        pltpu.VMEM((_BPW * _L,), jnp.float32),
        pltpu.SemaphoreType.DMA,
    ],
)
def _sc_logits(ctxidx_hbm, wout_hbm, hid_hbm, out_hbm,
               xidx_v, hid_v, ctx_v, out_v, sem):
    wid = lax.axis_index("s") * _NC + lax.axis_index("c")
    base = wid * _BPW
    pltpu.sync_copy(ctxidx_hbm.at[pl.ds(base * _L, _BPW * _L)], xidx_v)

    lanes = lax.iota(jnp.int32, 16)

    def fire(sc, hofs, cofs):
        pltpu.async_copy(
            hid_hbm.at[pl.ds((base + sc * _CB) * _EMB, _CB * _EMB)],
            hid_v.at[pl.ds(hofs * _EMB, _CB * _EMB)], sem)
        for p in range(_ROWS // 16):
            xv = xidx_v[pl.ds(sc * _ROWS + p * 16, 16)]
            for j in range(16):
                pltpu.async_copy(
                    wout_hbm.at[pl.ds(xv[j], 1)],
                    ctx_v.at[pl.ds(cofs + p * 16 + j, 1)], sem)

    fire(0, 0, 0)

    def sub(sc, _):
        p = sc % 2
        hofs = p * _CB
        cofs = p * _ROWS
        pltpu.make_async_copy(
            hid_hbm.at[pl.ds(0, _CB * _EMB)],
            hid_v.at[pl.ds(hofs * _EMB, _CB * _EMB)], sem).wait()
        pltpu.make_async_copy(
            wout_hbm.at[pl.ds(0, _ROWS)], ctx_v.at[pl.ds(cofs, _ROWS)],
            sem).wait()

        @pl.when(sc < _NSUB - 1)
        def _():
            q = (sc + 1) % 2
            fire(sc + 1, q * _CB, q * _ROWS)

        hbase = (lanes + hofs) * _EMB
        crow = lanes * _L + cofs
        obase = sc * _ROWS + lanes * _L

        def dstep(dc, accs):
            d0 = dc * _DC
            dcols = [jnp.full((16,), d0 + dd, jnp.int32)
                     for dd in range(_DC)]
            hvs = [plsc.load_gather(hid_v, [hbase + (d0 + dd)])
                   for dd in range(_DC)]
            out = []
            for l in range(_L):
                rowv = crow + l
                acc = accs[l]
                for dd in range(_DC):
                    cv = plsc.load_gather(ctx_v, [rowv, dcols[dd]])
                    acc = acc + cv * hvs[dd]
                out.append(acc)
            return tuple(out)

        accs = lax.fori_loop(
            0, _EMB // _DC, dstep,
            tuple(jnp.zeros(16, jnp.float32) for _ in range(_L)))

        for l in range(_L):
            sig = 1.0 / (1.0 + jnp.exp(-accs[l]))
            plsc.store_scatter(out_v, [obase + l], sig)
        return 0

    lax.fori_loop(0, _NSUB, sub, 0)
    pltpu.sync_copy(out_v, out_hbm.at[pl.ds(base * _L, _BPW * _L)])


def kernel(central_items, context_items, W_in, W_out):
    hidden = _sc_hidden(central_items.astype(jnp.int32), W_in.T)
    out = _sc_logits(
        context_items.reshape(-1).astype(jnp.int32),
        W_out,
        hidden,
    )
    return out.reshape(_B, _L)


# speedup vs baseline: 1.5477x; 1.0129x over previous
import functools

import jax
import jax.numpy as jnp
from jax import lax
from jax.experimental import pallas as pl
from jax.experimental.pallas import tpu as pltpu
from jax.experimental.pallas import tpu_sc as plsc

_EMB = 64
_B = 16384
_L = 20
_NC = 2
_NS = 16
_NW = _NC * _NS
_BPW = _B // _NW
_CB = 16
_NSUB = _BPW // _CB
_ROWS = _CB * _L
_DC = 8

_mesh = plsc.VectorSubcoreMesh(
    core_axis_name="c", subcore_axis_name="s", num_cores=_NC, num_subcores=_NS
)
_params = pltpu.CompilerParams(
    use_tc_tiling_on_sc=True, needs_layout_passes=False,
    disable_bounds_checks=True)


@functools.partial(
    pl.kernel,
    out_type=jax.ShapeDtypeStruct((_B * _EMB,), jnp.float32),
    mesh=_mesh,
    compiler_params=_params,
    scratch_types=[
        pltpu.VMEM((_BPW,), jnp.int32),
        pltpu.VMEM((4 * _EMB, 128), jnp.float32),
        pltpu.VMEM((_CB * _EMB,), jnp.float32),
        pltpu.SemaphoreType.DMA,
    ],
)
def _sc_hidden(central_hbm, wint_hbm, hid_hbm, cidx_v, blk_v, rows_v, sem):
    wid = lax.axis_index("s") * _NC + lax.axis_index("c")
    base = wid * _BPW
    pltpu.sync_copy(central_hbm.at[pl.ds(base, _BPW)], cidx_v)

    lanes = lax.iota(jnp.int32, 16)
    dvecs = [lanes + 16 * j for j in range(_EMB // 16)]

    def fire(v, bofs):
        blk = pl.multiple_of((v >> 7) << 7, 128)
        pltpu.async_copy(wint_hbm.at[:, pl.ds(blk, 128)],
                         blk_v.at[pl.ds(bofs, _EMB)], sem)

    first = cidx_v[pl.ds(0, 16)]
    for k in range(4):
        fire(first[k], k * _EMB)

    def group(g, _):
        vv = cidx_v[pl.ds(g * _CB, _CB)]
        nxt = cidx_v[pl.ds((g + 1) % _NSUB * _CB, _CB)]
        for j in range(_CB):
            slot = (g * _CB + j) % 4
            bofs = slot * _EMB
            pltpu.make_async_copy(
                wint_hbm.at[:, pl.ds(0, 128)], blk_v.at[pl.ds(bofs, _EMB)],
                sem).wait()
            if j + 4 < _CB:
                fire(vv[j + 4], bofs)
            else:
                @pl.when(g < _NSUB - 1)
                def _():
                    fire(nxt[j + 4 - _CB], bofs)
            m = jnp.full((16,), vv[j] & 127, jnp.int32)
            for j4 in range(_EMB // 16):
                col = plsc.load_gather(blk_v, [bofs + dvecs[j4], m])
                rows_v[pl.ds(j * _EMB + j4 * 16, 16)] = col
        pltpu.sync_copy(
            rows_v, hid_hbm.at[pl.ds((base + g * _CB) * _EMB, _CB * _EMB)])
        return 0

    lax.fori_loop(0, _NSUB, group, 0)


@functools.partial(
    pl.kernel,
    out_type=jax.ShapeDtypeStruct((_B * _L,), jnp.float32),
    mesh=_mesh,
    compiler_params=_params,
    scratch_types=[
        pltpu.VMEM((_BPW * _L,), jnp.int32),
        pltpu.VMEM((2 * _CB * _EMB,), jnp.float32),
        pltpu.VMEM((2 * _ROWS, _EMB), jnp.float32),
        pltpu.VMEM((_BPW * _L,), jnp.float32),
        pltpu.SemaphoreType.DMA,
    ],
)
def _sc_logits(ctxidx_hbm, wout_hbm, hid_hbm, out_hbm,
               xidx_v, hid_v, ctx_v, out_v, sem):
    wid = lax.axis_index("s") * _NC + lax.axis_index("c")
    base = wid * _BPW
    pltpu.sync_copy(ctxidx_hbm.at[pl.ds(base * _L, _BPW * _L)], xidx_v)

    lanes = lax.iota(jnp.int32, 16)

    def fire(sc, hofs, cofs):
        pltpu.async_copy(
            hid_hbm.at[pl.ds((base + sc * _CB) * _EMB, _CB * _EMB)],
            hid_v.at[pl.ds(hofs * _EMB, _CB * _EMB)], sem)
        for p in range(_ROWS // 16):
            xv = xidx_v[pl.ds(sc * _ROWS + p * 16, 16)]
            for j in range(16):
                pltpu.async_copy(
                    wout_hbm.at[pl.ds(xv[j], 1)],
                    ctx_v.at[pl.ds(cofs + p * 16 + j, 1)], sem)

    fire(0, 0, 0)

    def sub(sc, _):
        p = sc % 2
        hofs = p * _CB
        cofs = p * _ROWS
        @pl.when(sc < _NSUB - 1)
        def _():
            q = (sc + 1) % 2
            fire(sc + 1, q * _CB, q * _ROWS)

        pltpu.make_async_copy(
            hid_hbm.at[pl.ds(0, _CB * _EMB)],
            hid_v.at[pl.ds(hofs * _EMB, _CB * _EMB)], sem).wait()
        pltpu.make_async_copy(
            wout_hbm.at[pl.ds(0, _ROWS)], ctx_v.at[pl.ds(cofs, _ROWS)],
            sem).wait()

        hbase = (lanes + hofs) * _EMB
        crow = lanes * _L + cofs
        obase = sc * _ROWS + lanes * _L

        def dstep(dc, accs):
            d0 = dc * _DC
            dcols = [jnp.full((16,), d0 + dd, jnp.int32)
                     for dd in range(_DC)]
            hvs = [plsc.load_gather(hid_v, [hbase + (d0 + dd)])
                   for dd in range(_DC)]
            out = []
            for l in range(_L):
                rowv = crow + l
                acc = accs[l]
                for dd in range(_DC):
                    cv = plsc.load_gather(ctx_v, [rowv, dcols[dd]])
                    acc = acc + cv * hvs[dd]
                out.append(acc)
            return tuple(out)

        accs = lax.fori_loop(
            0, _EMB // _DC, dstep,
            tuple(jnp.zeros(16, jnp.float32) for _ in range(_L)))

        for l in range(_L):
            sig = 1.0 / (1.0 + jnp.exp(-accs[l]))
            plsc.store_scatter(out_v, [obase + l], sig)
        return 0

    lax.fori_loop(0, _NSUB, sub, 0)
    pltpu.sync_copy(out_v, out_hbm.at[pl.ds(base * _L, _BPW * _L)])


def kernel(central_items, context_items, W_in, W_out):
    hidden = _sc_hidden(central_items.astype(jnp.int32), W_in.T)
    out = _sc_logits(
        context_items.reshape(-1).astype(jnp.int32),
        W_out,
        hidden,
    )
    return out.reshape(_B, _L)


# speedup vs baseline: 1.5990x; 1.0331x over previous
import functools

import jax
import jax.numpy as jnp
from jax import lax
from jax.experimental import pallas as pl
from jax.experimental.pallas import tpu as pltpu
from jax.experimental.pallas import tpu_sc as plsc

_EMB = 64
_B = 16384
_L = 20
_NC = 2
_NS = 16
_NW = _NC * _NS
_BPW = _B // _NW
_CB = 16
_NSUB = _BPW // _CB
_ROWS = _CB * _L
_DC = 8

_mesh = plsc.VectorSubcoreMesh(
    core_axis_name="c", subcore_axis_name="s", num_cores=_NC, num_subcores=_NS
)
_params = pltpu.CompilerParams(
    use_tc_tiling_on_sc=True, needs_layout_passes=False,
    disable_bounds_checks=True)


@functools.partial(
    pl.kernel,
    out_type=jax.ShapeDtypeStruct((_B * _EMB,), jnp.float32),
    mesh=_mesh,
    compiler_params=_params,
    scratch_types=[
        pltpu.VMEM((_BPW,), jnp.int32),
        pltpu.VMEM((4 * _EMB, 128), jnp.float32),
        pltpu.VMEM((_CB * _EMB,), jnp.float32),
        pltpu.SemaphoreType.DMA,
    ],
)
def _sc_hidden(central_hbm, wint_hbm, hid_hbm, cidx_v, blk_v, rows_v, sem):
    wid = lax.axis_index("s") * _NC + lax.axis_index("c")
    base = wid * _BPW
    pltpu.sync_copy(central_hbm.at[pl.ds(base, _BPW)], cidx_v)

    lanes = lax.iota(jnp.int32, 16)
    dvecs = [lanes + 16 * j for j in range(_EMB // 16)]

    def fire(v, bofs):
        blk = pl.multiple_of((v >> 7) << 7, 128)
        pltpu.async_copy(wint_hbm.at[:, pl.ds(blk, 128)],
                         blk_v.at[pl.ds(bofs, _EMB)], sem)

    first = cidx_v[pl.ds(0, 16)]
    for k in range(4):
        fire(first[k], k * _EMB)

    def group(g, _):
        vv = cidx_v[pl.ds(g * _CB, _CB)]
        nxt = cidx_v[pl.ds((g + 1) % _NSUB * _CB, _CB)]
        for j in range(_CB):
            slot = (g * _CB + j) % 4
            bofs = slot * _EMB
            pltpu.make_async_copy(
                wint_hbm.at[:, pl.ds(0, 128)], blk_v.at[pl.ds(bofs, _EMB)],
                sem).wait()
            if j + 4 < _CB:
                fire(vv[j + 4], bofs)
            else:
                @pl.when(g < _NSUB - 1)
                def _():
                    fire(nxt[j + 4 - _CB], bofs)
            m = jnp.full((16,), vv[j] & 127, jnp.int32)
            for j4 in range(_EMB // 16):
                col = plsc.load_gather(blk_v, [bofs + dvecs[j4], m])
                rows_v[pl.ds(j * _EMB + j4 * 16, 16)] = col
        pltpu.sync_copy(
            rows_v, hid_hbm.at[pl.ds((base + g * _CB) * _EMB, _CB * _EMB)])
        return 0

    lax.fori_loop(0, _NSUB, group, 0)


@functools.partial(
    pl.kernel,
    out_type=jax.ShapeDtypeStruct((_B * _L,), jnp.float32),
    mesh=_mesh,
    compiler_params=_params,
    scratch_types=[
        pltpu.VMEM((_BPW * _L,), jnp.int32),
        pltpu.VMEM((2 * _CB * _EMB,), jnp.float32),
        pltpu.VMEM((2 * _ROWS, _EMB), jnp.float32),
        pltpu.VMEM((_BPW * _L,), jnp.float32),
        pltpu.SemaphoreType.DMA,
    ],
)
def _sc_logits(ctxidx_hbm, wout_hbm, hid_hbm, out_hbm,
               xidx_v, hid_v, ctx_v, out_v, sem):
    wid = lax.axis_index("s") * _NC + lax.axis_index("c")
    base = wid * _BPW
    pltpu.sync_copy(ctxidx_hbm.at[pl.ds(base * _L, _BPW * _L)], xidx_v)

    lanes = lax.iota(jnp.int32, 16)

    def fire(sc, hofs, cofs):
        pltpu.async_copy(
            hid_hbm.at[pl.ds((base + sc * _CB) * _EMB, _CB * _EMB)],
            hid_v.at[pl.ds(hofs * _EMB, _CB * _EMB)], sem)
        for p in range(_ROWS // 16):
            xv = xidx_v[pl.ds(sc * _ROWS + p * 16, 16)]
            for j in range(16):
                pltpu.async_copy(
                    wout_hbm.at[pl.ds(xv[j], 1)],
                    ctx_v.at[pl.ds(cofs + p * 16 + j, 1)], sem)

    fire(0, 0, 0)

    def sub(i, _):
      for half in range(2):
        sc = 2 * i + half
        hofs = half * _CB
        cofs = half * _ROWS
        @pl.when(sc < _NSUB - 1)
        def _():
            fire(sc + 1, (1 - half) * _CB, (1 - half) * _ROWS)

        pltpu.make_async_copy(
            hid_hbm.at[pl.ds(0, _CB * _EMB)],
            hid_v.at[pl.ds(hofs * _EMB, _CB * _EMB)], sem).wait()
        pltpu.make_async_copy(
            wout_hbm.at[pl.ds(0, _ROWS)], ctx_v.at[pl.ds(cofs, _ROWS)],
            sem).wait()

        hbase = (lanes + hofs) * _EMB
        crow = lanes * _L + cofs
        obase = sc * _ROWS + lanes * _L

        def dstep(dc, accs):
            d0 = dc * _DC
            dcols = [jnp.full((16,), d0 + dd, jnp.int32)
                     for dd in range(_DC)]
            hvs = [plsc.load_gather(hid_v, [hbase + (d0 + dd)])
                   for dd in range(_DC)]
            out = []
            for l in range(_L):
                rowv = crow + l
                acc = accs[l]
                for dd in range(_DC):
                    cv = plsc.load_gather(ctx_v, [rowv, dcols[dd]])
                    acc = acc + cv * hvs[dd]
                out.append(acc)
            return tuple(out)

        accs = lax.fori_loop(
            0, _EMB // _DC, dstep,
            tuple(jnp.zeros(16, jnp.float32) for _ in range(_L)))

        for l in range(_L):
            sig = 1.0 / (1.0 + jnp.exp(-accs[l]))
            plsc.store_scatter(out_v, [obase + l], sig)
      return 0

    lax.fori_loop(0, _NSUB // 2, sub, 0)
    pltpu.sync_copy(out_v, out_hbm.at[pl.ds(base * _L, _BPW * _L)])


def kernel(central_items, context_items, W_in, W_out):
    hidden = _sc_hidden(central_items.astype(jnp.int32), W_in.T)
    out = _sc_logits(
        context_items.reshape(-1).astype(jnp.int32),
        W_out,
        hidden,
    )
    return out.reshape(_B, _L)
